# baseline (speedup 1.0000x reference)
import functools

import jax
import jax.numpy as jnp
from jax import lax
from jax.experimental import pallas as pl
from jax.experimental.pallas import tpu as pltpu
from jax.experimental.pallas import tpu_sc as plsc

B = 32
T = 2048
D = 64
NC = 2
NS = 16
NW = NC * NS
P = T // NW
LANES = 16


def _emb_kernel(x_hbm, tok_hbm, pos_hbm, out_hbm, idx_v, pos_v, rows_v, sem):
    c = lax.axis_index("c")
    s = lax.axis_index("s")
    w = s * NC + c
    pbase = w * P

    pltpu.sync_copy(pos_hbm.at[pl.ds(pbase, P)], pos_v)

    @pl.loop(0, B)
    def _batch(b):
        pltpu.sync_copy(x_hbm.at[b, pl.ds(pbase, P)], idx_v)
        pltpu.async_copy(tok_hbm.at[idx_v], rows_v, sem).wait()

        @pl.loop(0, P)
        def _row(r):
            for d in range(D // LANES):
                sl = pl.ds(d * LANES, LANES)
                plsc.addupdate(rows_v.at[r, sl], pos_v[r, sl])

        pltpu.sync_copy(rows_v, out_hbm.at[b, pl.ds(pbase, P), :])


@jax.jit
def _emb(x, tok_table, pos_table):
    mesh = plsc.VectorSubcoreMesh(
        core_axis_name="c", subcore_axis_name="s", num_cores=NC,
        num_subcores=NS)
    return pl.kernel(
        _emb_kernel,
        out_type=jax.ShapeDtypeStruct((B, T, D), jnp.float32),
        mesh=mesh,
        compiler_params=pltpu.CompilerParams(use_tc_tiling_on_sc=False),
        scratch_types=[
            pltpu.VMEM((P,), jnp.int32),
            pltpu.VMEM((P, D), jnp.float32),
            pltpu.VMEM((P, D), jnp.float32),
            pltpu.SemaphoreType.DMA,
        ],
    )(x, tok_table, pos_table)


def kernel(x, tok_table, pos_table):
    return _emb(x.astype(jnp.int32), tok_table, pos_table)


# speedup vs baseline: 1.4874x; 1.4874x over previous
import functools

import jax
import jax.numpy as jnp
from jax import lax
from jax.experimental import pallas as pl
from jax.experimental.pallas import tpu as pltpu
from jax.experimental.pallas import tpu_sc as plsc

B = 32
T = 2048
D = 64
NC = 2
NS = 16
NW = NC * NS
SW = 128
NH = NW // (T // SW)
BH = B // NH
LANES = 16
NG = SW // LANES


def _emb_kernel(x_hbm, tok_hbm, pos_hbm, out_hbm,
                idx_blk, t_idx, r_v, rows_v, stage_v, sem):
    c = lax.axis_index("c")
    s = lax.axis_index("s")
    w = s * NC + c
    h = w // (T // SW)
    sidx = w % (T // SW)
    bbase = h * BH
    sbase = sidx * SW

    for blk in range(BH // 8):
        pltpu.sync_copy(
            x_hbm.at[pl.ds(bbase + blk * 8, 8), pl.ds(sbase, SW)], idx_blk)

        @pl.loop(0, 8)
        def _batch(bi):
            b = bbase + blk * 8 + bi
            for g in range(NG):
                sl = pl.ds(g * LANES, LANES)
                v = idx_blk[bi, sl]
                t_idx[sl] = lax.shift_right_logical(v, 3)
                r_v[sl] = lax.bitwise_and(v, 7)

            pltpu.sync_copy(pos_hbm.at[:, pl.ds(sbase, SW)], stage_v)

            for half in range(2):
                hb = half * (SW // 2)

                @pl.loop(0, SW // 2)
                def _fire(i):
                    u = t_idx[pl.ds(hb + i, LANES)][0]
                    pltpu.async_copy(tok_hbm.at[u], rows_v.at[i], sem)

                @pl.loop(0, SW // 2)
                def _drain(i):
                    pltpu.make_async_copy(
                        tok_hbm.at[0], rows_v.at[i], sem).wait()

                for g in range(NG // 2):
                    sl = pl.ds(hb + g * LANES, LANES)
                    i_vec = lax.iota(jnp.int32, LANES) + g * LANES
                    r_vec = r_v[sl]
                    for d in range(D):
                        d_vec = jnp.full((LANES,), d, jnp.int32)
                        vals = plsc.load_gather(rows_v, [i_vec, r_vec, d_vec])
                        plsc.addupdate(stage_v.at[d, sl], vals)
            pltpu.sync_copy(stage_v, out_hbm.at[b, :, pl.ds(sbase, SW)])


@jax.jit
def _emb(x, tok_table, pos_table):
    tok3 = tok_table.reshape(125000, 8, D)
    pos_t = pos_table.T
    mesh = plsc.VectorSubcoreMesh(
        core_axis_name="c", subcore_axis_name="s", num_cores=NC,
        num_subcores=NS)
    out_t = pl.kernel(
        _emb_kernel,
        out_type=jax.ShapeDtypeStruct((B, D, T), jnp.float32),
        mesh=mesh,
        compiler_params=pltpu.CompilerParams(
            use_tc_tiling_on_sc=True, needs_layout_passes=False),
        scratch_types=[
            pltpu.VMEM((8, SW), jnp.int32),
            pltpu.VMEM((SW + LANES,), jnp.int32),
            pltpu.VMEM((SW,), jnp.int32),
            pltpu.VMEM((SW // 2, 8, D), jnp.float32),
            pltpu.VMEM((D, SW), jnp.float32),
            pltpu.SemaphoreType.DMA,
        ],
    )(x, tok3, pos_t)
    return out_t.transpose(0, 2, 1)


def kernel(x, tok_table, pos_table):
    return _emb(x.astype(jnp.int32), tok_table, pos_table)


# speedup vs baseline: 1.6547x; 1.1125x over previous
import functools

import jax
import jax.numpy as jnp
from jax import lax
from jax.experimental import pallas as pl
from jax.experimental.pallas import tpu as pltpu
from jax.experimental.pallas import tpu_sc as plsc

B = 32
T = 2048
D = 64
NC = 2
NS = 16
NW = NC * NS
SW = 128
NH = NW // (T // SW)
BH = B // NH
LANES = 16
NG = SW // LANES
NCK = 4


def _emb_kernel(x_hbm, tok_hbm, pos_hbm, out_hbm,
                idx_blk, t_idx, r_v, rows_a, rows_b, stage_v,
                sem_a, sem_b, sem_p):
    c = lax.axis_index("c")
    s = lax.axis_index("s")
    w = s * NC + c
    h = w // (T // SW)
    sidx = w % (T // SW)
    bbase = h * BH
    sbase = sidx * SW

    for blk in range(BH // 8):
        pltpu.sync_copy(
            x_hbm.at[pl.ds(bbase + blk * 8, 8), pl.ds(sbase, SW)], idx_blk)

        @pl.loop(0, 8)
        def _batch(bi):
            b = bbase + blk * 8 + bi
            for g in range(NG):
                sl = pl.ds(g * LANES, LANES)
                v = idx_blk[bi, sl]
                t_idx[sl] = lax.shift_right_logical(v, 3)
                r_v[sl] = lax.bitwise_and(v, 7)

            pltpu.async_copy(pos_hbm.at[:, pl.ds(sbase, SW)], stage_v, sem_p)

            rows = (rows_a, rows_b)
            sems = (sem_a, sem_b)
            CH = SW // NCK

            def fire(ck):
                buf, sm = rows[ck % 2], sems[ck % 2]

                @pl.loop(0, CH)
                def _fire(i):
                    u = t_idx[pl.ds(ck * CH + i, LANES)][0]
                    pltpu.async_copy(tok_hbm.at[u], buf.at[i], sm)

            fire(0)
            for ck in range(NCK):
                if ck + 1 < NCK:
                    fire(ck + 1)
                if ck == 0:
                    pltpu.make_async_copy(
                        pos_hbm.at[:, pl.ds(sbase, SW)], stage_v, sem_p).wait()
                buf, sm = rows[ck % 2], sems[ck % 2]
                pltpu.make_async_copy(
                    tok_hbm.at[pl.ds(0, CH)], buf, sm).wait()
                for g in range(CH // LANES):
                    sl = pl.ds(ck * CH + g * LANES, LANES)
                    i_vec = lax.iota(jnp.int32, LANES) + g * LANES
                    r_vec = r_v[sl]
                    for d in range(D):
                        d_vec = jnp.full((LANES,), d, jnp.int32)
                        vals = plsc.load_gather(buf, [i_vec, r_vec, d_vec])
                        plsc.addupdate(stage_v.at[d, sl], vals)
            pltpu.sync_copy(stage_v, out_hbm.at[b, :, pl.ds(sbase, SW)])


@jax.jit
def _emb(x, tok_table, pos_table):
    tok3 = tok_table.reshape(125000, 8, D)
    pos_t = pos_table.T
    mesh = plsc.VectorSubcoreMesh(
        core_axis_name="c", subcore_axis_name="s", num_cores=NC,
        num_subcores=NS)
    out_t = pl.kernel(
        _emb_kernel,
        out_type=jax.ShapeDtypeStruct((B, D, T), jnp.float32),
        mesh=mesh,
        compiler_params=pltpu.CompilerParams(
            use_tc_tiling_on_sc=True, needs_layout_passes=False),
        scratch_types=[
            pltpu.VMEM((8, SW), jnp.int32),
            pltpu.VMEM((SW + LANES,), jnp.int32),
            pltpu.VMEM((SW,), jnp.int32),
            pltpu.VMEM((SW // NCK, 8, D), jnp.float32),
            pltpu.VMEM((SW // NCK, 8, D), jnp.float32),
            pltpu.VMEM((D, SW), jnp.float32),
            pltpu.SemaphoreType.DMA,
            pltpu.SemaphoreType.DMA,
            pltpu.SemaphoreType.DMA,
        ],
    )(x, tok3, pos_t)
    return out_t.transpose(0, 2, 1)


def kernel(x, tok_table, pos_table):
    return _emb(x.astype(jnp.int32), tok_table, pos_table)


# speedup vs baseline: 1.6815x; 1.0162x over previous
import functools

import jax
import jax.numpy as jnp
from jax import lax
from jax.experimental import pallas as pl
from jax.experimental.pallas import tpu as pltpu
from jax.experimental.pallas import tpu_sc as plsc

B = 32
T = 2048
D = 64
NC = 2
NS = 16
NW = NC * NS
SW = 128
NH = NW // (T // SW)
BH = B // NH
LANES = 16
NG = SW // LANES
NCK = 4


def _emb_kernel(x_hbm, tok_hbm, pos_hbm, out_hbm,
                idx_blk, t_idx, r_v, rows_a, rows_b, stage_v,
                sem_a, sem_b, sem_p):
    c = lax.axis_index("c")
    s = lax.axis_index("s")
    w = s * NC + c
    h = w // (T // SW)
    sidx = w % (T // SW)
    bbase = h * BH
    sbase = sidx * SW

    for blk in range(BH // 8):
        pltpu.sync_copy(
            x_hbm.at[pl.ds(bbase + blk * 8, 8), pl.ds(sbase, SW)], idx_blk)

        @pl.loop(0, 8)
        def _batch(bi):
            b = bbase + blk * 8 + bi
            for g in range(NG):
                sl = pl.ds(g * LANES, LANES)
                v = idx_blk[bi, sl]
                t_idx[sl] = lax.shift_right_logical(v, 3)
                r_v[sl] = lax.bitwise_and(v, 7)

            pltpu.async_copy(pos_hbm.at[:, pl.ds(sbase, SW)], stage_v, sem_p)

            rows = (rows_a, rows_b)
            sems = (sem_a, sem_b)
            CH = SW // NCK

            def fire(ck):
                buf, sm = rows[ck % 2], sems[ck % 2]

                @pl.loop(0, CH, unroll=4)
                def _fire(i):
                    u = t_idx[pl.ds(ck * CH + i, LANES)][0]
                    pltpu.async_copy(
                        tok_hbm.at[u], buf.at[pl.ds(i * 8, 8), :], sm)

            fire(0)
            for ck in range(NCK):
                if ck + 1 < NCK:
                    fire(ck + 1)
                if ck == 0:
                    pltpu.make_async_copy(
                        pos_hbm.at[:, pl.ds(sbase, SW)], stage_v, sem_p).wait()
                buf, sm = rows[ck % 2], sems[ck % 2]
                pltpu.make_async_copy(
                    tok_hbm.at[pl.ds(0, CH)],
                    buf.at[pl.ds(0, CH * 8), :], sm).wait()
                combs = []
                for g in range(CH // LANES):
                    sl = pl.ds(ck * CH + g * LANES, LANES)
                    i_vec = lax.iota(jnp.int32, LANES) + g * LANES
                    combs.append((sl, i_vec * 8 + r_v[sl]))
                for d in range(D):
                    d_vec = jnp.full((LANES,), d, jnp.int32)
                    for sl, comb in combs:
                        vals = plsc.load_gather(buf, [comb, d_vec])
                        plsc.addupdate(stage_v.at[d, sl], vals)
            pltpu.sync_copy(stage_v, out_hbm.at[b, :, pl.ds(sbase, SW)])


@jax.jit
def _emb(x, tok_table, pos_table):
    tok3 = tok_table.reshape(125000, 8, D)
    pos_t = pos_table.T
    mesh = plsc.VectorSubcoreMesh(
        core_axis_name="c", subcore_axis_name="s", num_cores=NC,
        num_subcores=NS)
    out_t = pl.kernel(
        _emb_kernel,
        out_type=jax.ShapeDtypeStruct((B, D, T), jnp.float32),
        mesh=mesh,
        compiler_params=pltpu.CompilerParams(
            use_tc_tiling_on_sc=True, needs_layout_passes=False),
        scratch_types=[
            pltpu.VMEM((8, SW), jnp.int32),
            pltpu.VMEM((SW + LANES,), jnp.int32),
            pltpu.VMEM((SW,), jnp.int32),
            pltpu.VMEM((SW // NCK * 8, D), jnp.float32),
            pltpu.VMEM((SW // NCK * 8, D), jnp.float32),
            pltpu.VMEM((D, SW), jnp.float32),
            pltpu.SemaphoreType.DMA,
            pltpu.SemaphoreType.DMA,
            pltpu.SemaphoreType.DMA,
        ],
    )(x, tok3, pos_t)
    return out_t.transpose(0, 2, 1)


def kernel(x, tok_table, pos_table):
    return _emb(x.astype(jnp.int32), tok_table, pos_table)
